# baseline (speedup 1.0000x reference)
import functools

import jax
import jax.numpy as jnp
from jax.experimental import pallas as pl
from jax.experimental.pallas import tpu as pltpu

_B_BLK = 8


def _rvq_body(x_ref, cb_ref, alpha_ref, out_ref, code_ref):
    alpha = alpha_ref[0, 0]
    S = x_ref.shape[2]
    kiota = jax.lax.broadcasted_iota(jnp.int32, (32, S), 0)

    prep = []
    for h in range(2):
        c = cb_ref[h]
        cn = jnp.sum(c * c, axis=1, keepdims=True)
        cs = -2.0 * c
        c2 = (1.0 - alpha) * c
        chi = c2.astype(jnp.bfloat16)
        clo = (c2 - chi.astype(jnp.float32)).astype(jnp.bfloat16)
        prep.append((cn, cs, chi, clo))

    for b in range(_B_BLK):
        x = x_ref[b]
        inds = []
        vs = []
        for h in range(2):
            cn, cs, chi, clo = prep[h]
            xh = x[h * 32:(h + 1) * 32]
            dotT = jax.lax.dot_general(
                cs, xh, (((1,), (0,)), ((), ())),
                precision=jax.lax.Precision.HIGHEST,
                preferred_element_type=jnp.float32)
            s = cn + dotT
            mins = jnp.min(s, axis=0, keepdims=True)
            ind = jnp.min(jnp.where(s <= mins, kiota, 32), axis=0)
            oh = (kiota == ind[None, :]).astype(jnp.bfloat16)
            v = jax.lax.dot_general(
                chi, oh, (((0,), (0,)), ((), ())),
                preferred_element_type=jnp.float32)
            v = v + jax.lax.dot_general(
                clo, oh, (((0,), (0,)), ((), ())),
                preferred_element_type=jnp.float32)
            vs.append(v)
            inds.append(ind)

        out_ref[b] = alpha * x + jnp.concatenate(vs, axis=0)
        code_ref[pl.ds(b * S, S)] = inds[0] + 32 * inds[1]


@functools.partial(jax.jit, static_argnames=())
def kernel(input, kernel, alpha):
    B, S, D = input.shape
    xt = jnp.transpose(input, (0, 2, 1))
    alpha_arr = jnp.asarray(alpha, jnp.float32).reshape(1, 1)
    grid = (B // _B_BLK,)
    out_t, code = pl.pallas_call(
        _rvq_body,
        grid=grid,
        in_specs=[
            pl.BlockSpec((_B_BLK, D, S), lambda i: (i, 0, 0)),
            pl.BlockSpec((2, 32, 32), lambda i: (0, 0, 0)),
            pl.BlockSpec(memory_space=pltpu.SMEM),
        ],
        out_specs=[
            pl.BlockSpec((_B_BLK, D, S), lambda i: (i, 0, 0)),
            pl.BlockSpec((_B_BLK * S,), lambda i: (i,)),
        ],
        out_shape=[
            jax.ShapeDtypeStruct((B, D, S), jnp.float32),
            jax.ShapeDtypeStruct((B * S,), jnp.int32),
        ],
    )(xt, kernel, alpha_arr)
    return jnp.transpose(out_t, (0, 2, 1)), code.reshape(B, S)


# speedup vs baseline: 1.1324x; 1.1324x over previous
import functools

import jax
import jax.numpy as jnp
from jax.experimental import pallas as pl
from jax.experimental.pallas import tpu as pltpu

_B_BLK = 8


def _rvq_body(x_ref, cs_ref, wg_ref, cnorm_ref, alpha_ref, out_ref, code_ref):
    cs = cs_ref[...]
    wg = wg_ref[...]
    cnorm = cnorm_ref[...]
    alpha = alpha_ref[0, 0]
    S = x_ref.shape[2]
    kiota = jax.lax.broadcasted_iota(jnp.int32, (32, S), 0)

    for b in range(_B_BLK):
        x = x_ref[b]

        dotT = jax.lax.dot_general(
            cs, x, (((1,), (0,)), ((), ())),
            precision=jax.lax.Precision.HIGHEST,
            preferred_element_type=jnp.float32)
        score = cnorm + dotT

        inds = []
        ohs = []
        for h in range(2):
            s = score[h * 32:(h + 1) * 32]
            mins = jnp.min(s, axis=0, keepdims=True)
            ind = jnp.min(jnp.where(s <= mins, kiota, 32), axis=0)
            ohs.append((kiota == ind[None, :]).astype(jnp.bfloat16))
            inds.append(ind)

        oh = jnp.concatenate(ohs, axis=0)
        r = jax.lax.dot_general(
            wg, oh, (((0,), (0,)), ((), ())),
            preferred_element_type=jnp.float32)
        out_ref[b] = alpha * x + (r[:64] + r[64:])
        code_ref[pl.ds(b * S, S)] = inds[0] + 32 * inds[1]


@functools.partial(jax.jit, static_argnames=())
def kernel(input, kernel, alpha):
    B, S, D = input.shape
    xt = jnp.transpose(input, (0, 2, 1))
    alpha_f = jnp.asarray(alpha, jnp.float32)
    alpha_arr = alpha_f.reshape(1, 1)
    cblk = jnp.zeros((2 * 32, D), jnp.float32)
    cblk = cblk.at[:32, :32].set(kernel[0]).at[32:, 32:].set(kernel[1])
    cnorm = jnp.sum(cblk * cblk, axis=1, keepdims=True)
    cs = -2.0 * cblk
    c2 = (1.0 - alpha_f) * cblk
    chi = c2.astype(jnp.bfloat16)
    clo = (c2 - chi.astype(jnp.float32)).astype(jnp.bfloat16)
    wg = jnp.concatenate([chi, clo], axis=1)
    grid = (B // _B_BLK,)
    out_t, code = pl.pallas_call(
        _rvq_body,
        grid=grid,
        in_specs=[
            pl.BlockSpec((_B_BLK, D, S), lambda i: (i, 0, 0)),
            pl.BlockSpec((64, D), lambda i: (0, 0)),
            pl.BlockSpec((64, 2 * D), lambda i: (0, 0)),
            pl.BlockSpec((64, 1), lambda i: (0, 0)),
            pl.BlockSpec(memory_space=pltpu.SMEM),
        ],
        out_specs=[
            pl.BlockSpec((_B_BLK, D, S), lambda i: (i, 0, 0)),
            pl.BlockSpec((_B_BLK * S,), lambda i: (i,)),
        ],
        out_shape=[
            jax.ShapeDtypeStruct((B, D, S), jnp.float32),
            jax.ShapeDtypeStruct((B * S,), jnp.int32),
        ],
    )(xt, cs, wg, cnorm, alpha_arr)
    return jnp.transpose(out_t, (0, 2, 1)), code.reshape(B, S)


# speedup vs baseline: 1.2950x; 1.1435x over previous
import functools

import jax
import jax.numpy as jnp
from jax.experimental import pallas as pl
from jax.experimental.pallas import tpu as pltpu

_B_BLK = 8


def _rvq_body(x_ref, cb_ref, alpha_ref, out_ref, code_ref):
    alpha = alpha_ref[0, 0]
    S = x_ref.shape[2]
    kiota = jax.lax.broadcasted_iota(jnp.int32, (32, S), 0)

    c0 = cb_ref[0]
    c1 = cb_ref[1]
    z = jnp.zeros((32, 32), jnp.float32)
    cblk = jnp.concatenate(
        [jnp.concatenate([c0, z], axis=1),
         jnp.concatenate([z, c1], axis=1)], axis=0)
    cnorm = jnp.sum(cblk * cblk, axis=1, keepdims=True)
    cs = -2.0 * cblk
    c2 = (1.0 - alpha) * cblk
    chi = c2.astype(jnp.bfloat16)
    clo = (c2 - chi.astype(jnp.float32)).astype(jnp.bfloat16)
    wg = jnp.concatenate([chi, clo], axis=1)

    for b in range(_B_BLK):
        x = x_ref[b]

        dotT = jax.lax.dot_general(
            cs, x, (((1,), (0,)), ((), ())),
            precision=jax.lax.Precision.HIGHEST,
            preferred_element_type=jnp.float32)
        score = cnorm + dotT

        inds = []
        ohs = []
        for h in range(2):
            s = score[h * 32:(h + 1) * 32]
            mins = jnp.min(s, axis=0, keepdims=True)
            ind = jnp.min(jnp.where(s <= mins, kiota, 32), axis=0)
            ohs.append((kiota == ind[None, :]).astype(jnp.bfloat16))
            inds.append(ind)

        oh = jnp.concatenate(ohs, axis=0)
        r = jax.lax.dot_general(
            wg, oh, (((0,), (0,)), ((), ())),
            preferred_element_type=jnp.float32)
        out_ref[b] = alpha * x + (r[:64] + r[64:])
        code_ref[b, :] = inds[0] + 32 * inds[1]


@functools.partial(jax.jit, static_argnames=())
def kernel(input, kernel, alpha):
    B, S, D = input.shape
    xt = jnp.transpose(input, (0, 2, 1))
    alpha_arr = jnp.asarray(alpha, jnp.float32).reshape(1, 1)
    grid = (B // _B_BLK,)
    out_t, code = pl.pallas_call(
        _rvq_body,
        grid=grid,
        in_specs=[
            pl.BlockSpec((_B_BLK, D, S), lambda i: (i, 0, 0)),
            pl.BlockSpec((2, 32, 32), lambda i: (0, 0, 0)),
            pl.BlockSpec(memory_space=pltpu.SMEM),
        ],
        out_specs=[
            pl.BlockSpec((_B_BLK, D, S), lambda i: (i, 0, 0)),
            pl.BlockSpec((_B_BLK, S), lambda i: (i, 0)),
        ],
        out_shape=[
            jax.ShapeDtypeStruct((B, D, S), jnp.float32),
            jax.ShapeDtypeStruct((B, S), jnp.int32),
        ],
    )(xt, kernel, alpha_arr)
    return jnp.transpose(out_t, (0, 2, 1)), code
